# Initial kernel scaffold; baseline (speedup 1.0000x reference)
import jax
import jax.numpy as jnp
from jax.experimental import pallas as pl


def kernel(x, edge_index, edge_attr, batch, W_enc, b_enc, edge_emb, eps, W1, b1, g1, bb1, W2, b2, gL, bL, Wp1, bp1, gp, bpb, Wp2, bp2):
    raise NotImplementedError("write your pallas kernel here")


# speedup vs baseline: 6.6066x; 6.6066x over previous
import functools

import jax
import jax.numpy as jnp
from jax import lax
from jax.experimental import pallas as pl
from jax.experimental.pallas import tpu as pltpu
from jax.experimental.pallas import tpu_sc as plsc

N = 10000
E = 320000
EMB = 128
HID = 2 * EMB
NLAYER = 5
G = 128
NT = 3
NET = 4

RB = 2000
NRB = N // RB

CHUNK = 128
NCHUNK = E // CHUNK
CHUNK_PER_SC = NCHUNK // 2
STRIPE = 624


def _relu(v):
    return jnp.maximum(v, 0.0)


def _enc_body(x_ref, We_ref, be_ref, ee_ref, h_ref, M_ref):
    h = jnp.dot(x_ref[...], We_ref[...], preferred_element_type=jnp.float32)
    h = h + be_ref[...]
    h_ref[...] = h
    for t in range(NET):
        M_ref[t] = _relu(h + ee_ref[pl.ds(t, 1), :])


@jax.jit
def _encoder(x, W_enc, be, ee0):
    return pl.pallas_call(
        _enc_body,
        grid=(NRB,),
        in_specs=[
            pl.BlockSpec((RB, 4), lambda i: (i, 0)),
            pl.BlockSpec((4, EMB), lambda i: (0, 0)),
            pl.BlockSpec((1, EMB), lambda i: (0, 0)),
            pl.BlockSpec((NET, EMB), lambda i: (0, 0)),
        ],
        out_specs=[
            pl.BlockSpec((RB, EMB), lambda i: (i, 0)),
            pl.BlockSpec((NET, RB, EMB), lambda i: (0, i, 0)),
        ],
        out_shape=[
            jax.ShapeDtypeStruct((N, EMB), jnp.float32),
            jax.ShapeDtypeStruct((NET, N, EMB), jnp.float32),
        ],
    )(x, W_enc, be, ee0)


def _cidx_body(src_ref, attr_ref, cidx_ref):
    cidx_ref[...] = attr_ref[...] * N + src_ref[...]


@jax.jit
def _cidx(src2d, attr2d):
    return pl.pallas_call(
        _cidx_body,
        in_specs=[pl.BlockSpec(src2d.shape, lambda: (0, 0))] * 2,
        out_specs=pl.BlockSpec(src2d.shape, lambda: (0, 0)),
        out_shape=jax.ShapeDtypeStruct(src2d.shape, jnp.int32),
    )(src2d, attr2d)


def _sc_agg_body(M_hbm, cidx_hbm, dst_hbm, zeros_hbm, out_hbm,
                 acc, idx_v, dst_v, rows_v, sem):
    c = lax.axis_index("c")
    s = lax.axis_index("s")

    @pl.when(s < 15)
    def _():
        pltpu.sync_copy(zeros_hbm.at[pl.ds(s * STRIPE, STRIPE)],
                        acc.at[pl.ds(s * STRIPE, STRIPE)])

    @pl.when(s == 15)
    def _():
        pltpu.sync_copy(zeros_hbm.at[pl.ds(15 * STRIPE, N - 15 * STRIPE)],
                        acc.at[pl.ds(15 * STRIPE, N - 15 * STRIPE)])

    plsc.subcore_barrier()

    nchunks = 78 + jnp.where(s < 2, 1, 0)

    def body(i, carry):
        chunk = c * CHUNK_PER_SC + s + 16 * i
        base = chunk * CHUNK
        pltpu.sync_copy(cidx_hbm.at[pl.ds(base, CHUNK)], idx_v)
        pltpu.sync_copy(dst_hbm.at[pl.ds(base, CHUNK)], dst_v)
        pltpu.async_copy(M_hbm.at[idx_v], rows_v, sem).wait()
        pltpu.sync_copy(rows_v, acc.at[dst_v], add=True)
        return carry

    lax.fori_loop(0, nchunks, body, 0)
    plsc.subcore_barrier()

    @pl.when(s < 15)
    def _():
        pltpu.sync_copy(acc.at[pl.ds(s * STRIPE, STRIPE)],
                        out_hbm.at[c, pl.ds(s * STRIPE, STRIPE)])

    @pl.when(s == 15)
    def _():
        pltpu.sync_copy(acc.at[pl.ds(15 * STRIPE, N - 15 * STRIPE)],
                        out_hbm.at[c, pl.ds(15 * STRIPE, N - 15 * STRIPE)])


@jax.jit
def _sc_agg(M_flat, cidx, dst, zeros):
    mesh = plsc.VectorSubcoreMesh(core_axis_name="c", subcore_axis_name="s")
    f = functools.partial(
        pl.kernel,
        mesh=mesh,
        out_type=jax.ShapeDtypeStruct((2, N, EMB), jnp.float32),
        scratch_types=[
            pltpu.VMEM_SHARED((N, EMB), jnp.float32),
            pltpu.VMEM((CHUNK,), jnp.int32),
            pltpu.VMEM((CHUNK,), jnp.int32),
            pltpu.VMEM((CHUNK, EMB), jnp.float32),
            pltpu.SemaphoreType.DMA,
        ],
    )(_sc_agg_body)
    return f(M_flat, cidx, dst, zeros)


def _layer_body(last, h_ref, parts_ref, W1_ref, b1_ref, g1_ref, bb1_ref,
                W2_ref, b2_ref, gL_ref, bL_ref, eps_ref, ee_ref,
                hn_ref, *maybe_M):
    h = h_ref[...]
    agg = parts_ref[0] + parts_ref[1]
    z = (1.0 + eps_ref[0, 0]) * h + agg
    z = jnp.dot(z, W1_ref[...], preferred_element_type=jnp.float32) + b1_ref[...]
    z = g1_ref[...] * z + bb1_ref[...]
    z = _relu(z)
    z = jnp.dot(z, W2_ref[...], preferred_element_type=jnp.float32) + b2_ref[...]
    z = gL_ref[...] * z + bL_ref[...]
    if not last:
        z = _relu(z)
    hn = h + z
    hn_ref[...] = hn
    if not last:
        M_ref = maybe_M[0]
        for t in range(NET):
            M_ref[t] = _relu(hn + ee_ref[pl.ds(t, 1), :])


@functools.partial(jax.jit, static_argnums=(0,))
def _layer(last, h, parts, W1l, b1l, g1l, bb1l, W2l, b2l, gLl, bLl, epsl, eenext):
    out_specs = [pl.BlockSpec((RB, EMB), lambda i: (i, 0))]
    out_shape = [jax.ShapeDtypeStruct((N, EMB), jnp.float32)]
    if not last:
        out_specs.append(pl.BlockSpec((NET, RB, EMB), lambda i: (0, i, 0)))
        out_shape.append(jax.ShapeDtypeStruct((NET, N, EMB), jnp.float32))
    res = pl.pallas_call(
        functools.partial(_layer_body, last),
        grid=(NRB,),
        in_specs=[
            pl.BlockSpec((RB, EMB), lambda i: (i, 0)),
            pl.BlockSpec((2, RB, EMB), lambda i: (0, i, 0)),
            pl.BlockSpec((EMB, HID), lambda i: (0, 0)),
            pl.BlockSpec((1, HID), lambda i: (0, 0)),
            pl.BlockSpec((1, HID), lambda i: (0, 0)),
            pl.BlockSpec((1, HID), lambda i: (0, 0)),
            pl.BlockSpec((HID, EMB), lambda i: (0, 0)),
            pl.BlockSpec((1, EMB), lambda i: (0, 0)),
            pl.BlockSpec((1, EMB), lambda i: (0, 0)),
            pl.BlockSpec((1, EMB), lambda i: (0, 0)),
            pl.BlockSpec((1, 1), lambda i: (0, 0)),
            pl.BlockSpec((NET, EMB), lambda i: (0, 0)),
        ],
        out_specs=out_specs,
        out_shape=out_shape,
    )(h, parts, W1l, b1l, g1l, bb1l, W2l, b2l, gLl, bLl, epsl, eenext)
    return res


def _pool_body(h_ref, b_ref, Wp1_ref, bp1_ref, gp_ref, bpb_ref, Wp2_ref,
               bp2_ref, out_ref, sums_ref, cnt_ref):
    i = pl.program_id(0)

    @pl.when(i == 0)
    def _():
        sums_ref[...] = jnp.zeros_like(sums_ref)
        cnt_ref[...] = jnp.zeros_like(cnt_ref)

    gid = lax.broadcasted_iota(jnp.int32, (RB, G), 1)
    P = (b_ref[...] == gid).astype(jnp.float32)
    sums_ref[...] += jax.lax.dot_general(
        P, h_ref[...], (((0,), (0,)), ((), ())),
        preferred_element_type=jnp.float32)
    cnt_ref[...] += jax.lax.dot_general(
        P, jnp.ones((RB, 1), jnp.float32), (((0,), (0,)), ((), ())),
        preferred_element_type=jnp.float32)

    @pl.when(i == NRB - 1)
    def _():
        hg = sums_ref[...] / jnp.maximum(cnt_ref[...], 1.0)
        p = jnp.dot(hg, Wp1_ref[...], preferred_element_type=jnp.float32)
        p = p + bp1_ref[...]
        p = _relu(gp_ref[...] * p + bpb_ref[...])
        out_ref[...] = jnp.dot(p, Wp2_ref[...],
                               preferred_element_type=jnp.float32) + bp2_ref[...]


@jax.jit
def _pool_pred(h, batch2d, Wp1, bp1, gp, bpb, Wp2, bp2):
    return pl.pallas_call(
        _pool_body,
        grid=(NRB,),
        in_specs=[
            pl.BlockSpec((RB, EMB), lambda i: (i, 0)),
            pl.BlockSpec((RB, 1), lambda i: (i, 0)),
            pl.BlockSpec((EMB, HID), lambda i: (0, 0)),
            pl.BlockSpec((1, HID), lambda i: (0, 0)),
            pl.BlockSpec((1, HID), lambda i: (0, 0)),
            pl.BlockSpec((1, HID), lambda i: (0, 0)),
            pl.BlockSpec((HID, NT), lambda i: (0, 0)),
            pl.BlockSpec((1, NT), lambda i: (0, 0)),
        ],
        out_specs=pl.BlockSpec((G, NT), lambda i: (0, 0)),
        out_shape=jax.ShapeDtypeStruct((G, NT), jnp.float32),
        scratch_shapes=[
            pltpu.VMEM((G, EMB), jnp.float32),
            pltpu.VMEM((G, 1), jnp.float32),
        ],
    )(h, batch2d, Wp1, bp1, gp, bpb, Wp2, bp2)


def kernel(x, edge_index, edge_attr, batch, W_enc, b_enc, edge_emb, eps,
           W1, b1, g1, bb1, W2, b2, gL, bL, Wp1, bp1, gp, bpb, Wp2, bp2):
    src2d = edge_index[0].reshape(E // 128, 128)
    attr2d = edge_attr.reshape(E // 128, 128)
    dst = edge_index[1]
    zeros = jnp.zeros((N, EMB), jnp.float32)

    h, M = _encoder(x, W_enc, b_enc.reshape(1, EMB), edge_emb[0])
    cidx = _cidx(src2d, attr2d).reshape(E)
    for l in range(NLAYER):
        parts = _sc_agg(M.reshape(NET * N, EMB), cidx, dst, zeros)
        last = l == NLAYER - 1
        eenext = edge_emb[l + 1] if not last else edge_emb[0]
        res = _layer(last, h, parts, W1[l], b1[l].reshape(1, HID),
                     g1[l].reshape(1, HID), bb1[l].reshape(1, HID),
                     W2[l], b2[l].reshape(1, EMB), gL[l].reshape(1, EMB),
                     bL[l].reshape(1, EMB), eps[l].reshape(1, 1), eenext)
        if last:
            (h,) = res
        else:
            h, M = res

    return _pool_pred(h, batch.reshape(N, 1), Wp1, bp1.reshape(1, HID),
                      gp.reshape(1, HID), bpb.reshape(1, HID), Wp2,
                      bp2.reshape(1, NT))
